# baseline (speedup 1.0000x reference)
import jax
import jax.numpy as jnp
from jax import lax
from jax.experimental import pallas as pl
from jax.experimental.pallas import tpu as pltpu

IN_C = 128
HEADS = 2
RATIO = 16
NBATCH = 8
NDST = NBATCH * RATIO
HC = HEADS * IN_C
HR = HEADS * RATIO
HD = HEADS * NDST
NEG_SLOPE = 0.2
TILE_N = 2048
MASKPOS = 3e38
MFLOOR = -1e33


def _bipart_pool_kernel(x_ref, batch_ref, xcb_ref, wl_ref, bl_ref, wr_ref,
                        br_ref, attbd_ref, s2n_ref, til2_ref, rep2_ref,
                        biasT_ref, out_ref, m_s, den_s, numT_s):
    f32 = jnp.float32
    t = pl.program_id(0)
    ntiles = pl.num_programs(0)
    rows = pl.ds(t * TILE_N, TILE_N)

    xs = x_ref[rows, :]
    batch_t = batch_ref[rows, :]
    xl = jnp.dot(xs, wl_ref[...], preferred_element_type=f32) + bl_ref[...]
    xr = (jnp.dot(xcb_ref[...], wr_ref[...], preferred_element_type=f32)
          + br_ref[...])

    onehot = (batch_t == lax.broadcasted_iota(jnp.int32, (1, NBATCH), 1)
              ).astype(f32)
    ohrep = jnp.dot(onehot, rep2_ref[...], preferred_element_type=f32)

    u = jnp.dot(xl, attbd_ref[...], preferred_element_type=f32)
    vr = jnp.dot(xr, attbd_ref[...], preferred_element_type=f32)
    ind = (lax.broadcasted_iota(jnp.int32, (RATIO, HR), 1) % RATIO ==
           lax.broadcasted_iota(jnp.int32, (RATIO, HR), 0)).astype(f32)
    v = jnp.sum(vr * ind, axis=0, keepdims=True)

    logc = u + v
    for r in range(RATIO):
        nz = jnp.minimum(xl + xr[r:r + 1, :], 0.0)
        logc = logc + jnp.dot(nz, s2n_ref[:, r * HR:(r + 1) * HR],
                              preferred_element_type=f32)
    l2 = (jnp.dot(logc, til2_ref[...], preferred_element_type=f32) * ohrep
          + (ohrep - 1.0) * MASKPOS)
    m_t = jnp.maximum(jnp.max(l2, axis=0, keepdims=True), MFLOOR)

    @pl.when(t == 0)
    def _():
        p = jnp.exp(l2 - m_t)
        m_s[0:1, :] = m_t
        den_s[0:1, :] = jnp.sum(p, axis=0, keepdims=True)
        numT_s[...] = lax.dot_general(xl, p, (((0,), (0,)), ((), ())),
                                      preferred_element_type=f32)

    @pl.when(t > 0)
    def _():
        m_old = m_s[0:1, :]
        m_new = jnp.maximum(m_old, m_t)
        corr = jnp.exp(m_old - m_new)
        p = jnp.exp(l2 - m_new)
        m_s[0:1, :] = m_new
        den_s[0:1, :] = den_s[0:1, :] * corr + jnp.sum(p, axis=0, keepdims=True)
        numT_s[...] = (numT_s[...] * corr
                       + lax.dot_general(xl, p, (((0,), (0,)), ((), ())),
                                         preferred_element_type=f32))

    @pl.when(t == ntiles - 1)
    def _finalize():
        acc = jnp.zeros((IN_C, NDST), f32)
        for h in range(HEADS):
            blk = numT_s[h * IN_C:(h + 1) * IN_C, h * NDST:(h + 1) * NDST]
            acc = acc + blk / (den_s[0:1, h * NDST:(h + 1) * NDST] + 1e-16)
        out_ref[...] = acc * (1.0 / HEADS) + biasT_ref[...]


def kernel(x, edge_index, batch, xcent_base, W_l, b_l, W_r, b_r, att, bias):
    del edge_index
    n = x.shape[0]
    n_pad = -(-n // TILE_N) * TILE_N
    ntiles = n_pad // TILE_N
    xp = jnp.pad(x, ((0, n_pad - n), (0, 0)))
    bp = jnp.pad(batch.astype(jnp.int32), (0, n_pad - n),
                 constant_values=NBATCH).reshape(n_pad, 1)

    f32 = jnp.float32
    cfull = jnp.arange(HC)
    jidx = jnp.arange(HR)
    d2 = jnp.arange(HD)
    attfull = att.reshape(HC)
    attbd = jnp.where((jidx[None, :] // RATIO) == (cfull[:, None] // IN_C),
                      attfull[:, None], 0.0).astype(f32)
    s2n = (-(1.0 - NEG_SLOPE) * attbd[:, None, :] *
           ((jidx[None, None, :] % RATIO) ==
            jnp.arange(RATIO)[None, :, None])).reshape(HC, RATIO * HR)
    til2 = (((d2[None, :] // NDST) == (jidx[:, None] // RATIO)) &
            ((d2[None, :] % RATIO) == (jidx[:, None] % RATIO))).astype(f32)
    rep2 = ((d2[None, :] % NDST) // RATIO ==
            jnp.arange(NBATCH)[:, None]).astype(f32)

    outT = pl.pallas_call(
        _bipart_pool_kernel,
        grid=(ntiles,),
        out_shape=jax.ShapeDtypeStruct((IN_C, NDST), jnp.float32),
        scratch_shapes=[
            pltpu.VMEM((1, HD), jnp.float32),
            pltpu.VMEM((1, HD), jnp.float32),
            pltpu.VMEM((HC, HD), jnp.float32),
        ],
    )(xp, bp, xcent_base, W_l, b_l.reshape(1, HC), W_r, br_row(b_r),
      attbd, s2n, til2, rep2, bias.reshape(IN_C, 1))
    return outT.T.reshape(NBATCH, RATIO, IN_C)


def br_row(b_r):
    return b_r.reshape(1, HC)


# speedup vs baseline: 1.2480x; 1.2480x over previous
import jax
import jax.numpy as jnp
from jax import lax
from jax.experimental import pallas as pl
from jax.experimental.pallas import tpu as pltpu

IN_C = 128
HEADS = 2
RATIO = 16
NBATCH = 8
NDST = NBATCH * RATIO
HC = HEADS * IN_C
HR = HEADS * RATIO
HD = HEADS * NDST
NEG_SLOPE = 0.2
MASKPOS = 3e38
MFLOOR = -1e33


def _bipart_pool_kernel(x_ref, batch_ref, xcb_ref, wl_ref, bl_ref, wr_ref,
                        br_ref, attc_ref, biasT_ref, out_ref,
                        attbd_s, s2n_s, til2_s, rep2_s, m_s, den_s, numT_s):
    f32 = jnp.float32
    t = pl.program_id(0)
    ntiles = pl.num_programs(0)

    @pl.when(t == 0)
    def _build_selectors():
        attc = attc_ref[...]
        cf = lax.broadcasted_iota(jnp.int32, (HC, HR), 0)
        jj = lax.broadcasted_iota(jnp.int32, (HC, HR), 1)
        attbd_s[...] = jnp.where(jj // RATIO == cf // IN_C, attc, 0.0)
        cf2 = lax.broadcasted_iota(jnp.int32, (HC, RATIO * HR), 0)
        kk = lax.broadcasted_iota(jnp.int32, (HC, RATIO * HR), 1)
        s2n_s[...] = jnp.where(
            kk % HR == (cf2 // IN_C) * RATIO + kk // HR,
            attc * (-(1.0 - NEG_SLOPE)), 0.0)
        j2 = lax.broadcasted_iota(jnp.int32, (HR, HD), 0)
        d2 = lax.broadcasted_iota(jnp.int32, (HR, HD), 1)
        til2_s[...] = jnp.where(
            (d2 // NDST == j2 // RATIO) & (d2 % RATIO == j2 % RATIO), 1.0, 0.0)
        bb = lax.broadcasted_iota(jnp.int32, (NBATCH, HD), 0)
        d3 = lax.broadcasted_iota(jnp.int32, (NBATCH, HD), 1)
        rep2_s[...] = jnp.where(d3 % NDST // RATIO == bb, 1.0, 0.0)

    xs = x_ref[...]
    batch_t = batch_ref[...]
    xl = jnp.dot(xs, wl_ref[...], preferred_element_type=f32) + bl_ref[...]
    xr = (jnp.dot(xcb_ref[...], wr_ref[...], preferred_element_type=f32)
          + br_ref[...])

    onehot = (batch_t == lax.broadcasted_iota(jnp.int32, (1, NBATCH), 1)
              ).astype(f32)
    ohrep = jnp.dot(onehot, rep2_s[...], preferred_element_type=f32)

    u = jnp.dot(xl, attbd_s[...], preferred_element_type=f32)
    vr = jnp.dot(xr, attbd_s[...], preferred_element_type=f32)
    ind = (lax.broadcasted_iota(jnp.int32, (RATIO, HR), 1) % RATIO ==
           lax.broadcasted_iota(jnp.int32, (RATIO, HR), 0)).astype(f32)
    v = jnp.sum(vr * ind, axis=0, keepdims=True)

    logc = u + v
    for r in range(RATIO):
        nz = jnp.minimum(xl + xr[r:r + 1, :], 0.0)
        logc = logc + jnp.dot(nz, s2n_s[:, r * HR:(r + 1) * HR],
                              preferred_element_type=f32)
    l2 = (jnp.dot(logc, til2_s[...], preferred_element_type=f32) * ohrep
          + (ohrep - 1.0) * MASKPOS)
    m_t = jnp.maximum(jnp.max(l2, axis=0, keepdims=True), MFLOOR)

    @pl.when(t == 0)
    def _():
        p = jnp.exp(l2 - m_t)
        m_s[...] = m_t
        den_s[...] = jnp.sum(p, axis=0, keepdims=True)
        numT_s[...] = lax.dot_general(xl, p, (((0,), (0,)), ((), ())),
                                      preferred_element_type=f32)

    @pl.when(t > 0)
    def _():
        m_old = m_s[...]
        m_new = jnp.maximum(m_old, m_t)
        corr = jnp.exp(m_old - m_new)
        p = jnp.exp(l2 - m_new)
        m_s[...] = m_new
        den_s[...] = den_s[...] * corr + jnp.sum(p, axis=0, keepdims=True)
        numT_s[...] = (numT_s[...] * corr
                       + lax.dot_general(xl, p, (((0,), (0,)), ((), ())),
                                         preferred_element_type=f32))

    @pl.when(t == ntiles - 1)
    def _finalize():
        acc = jnp.zeros((IN_C, NDST), f32)
        for h in range(HEADS):
            blk = numT_s[h * IN_C:(h + 1) * IN_C, h * NDST:(h + 1) * NDST]
            acc = acc + blk / (den_s[0:1, h * NDST:(h + 1) * NDST] + 1e-16)
        out_ref[...] = jnp.transpose(acc * (1.0 / HEADS) + biasT_ref[...])


def _run(xp, bp, xcent_base, W_l, b_l, W_r, b_r, att, bias, tile_n):
    ntiles = xp.shape[0] // tile_n
    return pl.pallas_call(
        _bipart_pool_kernel,
        grid=(ntiles,),
        in_specs=[
            pl.BlockSpec((tile_n, IN_C), lambda t: (t, 0)),
            pl.BlockSpec((tile_n, 1), lambda t: (t, 0)),
            pl.BlockSpec((RATIO, IN_C), lambda t: (0, 0)),
            pl.BlockSpec((IN_C, HC), lambda t: (0, 0)),
            pl.BlockSpec((1, HC), lambda t: (0, 0)),
            pl.BlockSpec((IN_C, HC), lambda t: (0, 0)),
            pl.BlockSpec((1, HC), lambda t: (0, 0)),
            pl.BlockSpec((HC, 1), lambda t: (0, 0)),
            pl.BlockSpec((IN_C, 1), lambda t: (0, 0)),
        ],
        out_specs=pl.BlockSpec((NDST, IN_C), lambda t: (0, 0)),
        out_shape=jax.ShapeDtypeStruct((NDST, IN_C), jnp.float32),
        scratch_shapes=[
            pltpu.VMEM((HC, HR), jnp.float32),
            pltpu.VMEM((HC, RATIO * HR), jnp.float32),
            pltpu.VMEM((HR, HD), jnp.float32),
            pltpu.VMEM((NBATCH, HD), jnp.float32),
            pltpu.VMEM((1, HD), jnp.float32),
            pltpu.VMEM((1, HD), jnp.float32),
            pltpu.VMEM((HC, HD), jnp.float32),
        ],
    )(xp, bp, xcent_base, W_l, b_l.reshape(1, HC), W_r, b_r.reshape(1, HC),
      att.reshape(HC, 1), bias.reshape(IN_C, 1))


def kernel(x, edge_index, batch, xcent_base, W_l, b_l, W_r, b_r, att, bias):
    del edge_index
    n = x.shape[0]
    if n % 2000 == 0:
        out = _run(x, batch.astype(jnp.int32).reshape(n, 1), xcent_base,
                   W_l, b_l, W_r, b_r, att, bias, 2000)
    else:
        n_pad = -(-n // 1024) * 1024
        xp = jnp.pad(x, ((0, n_pad - n), (0, 0)))
        bp = jnp.pad(batch.astype(jnp.int32), (0, n_pad - n),
                     constant_values=NBATCH).reshape(n_pad, 1)
        out = _run(xp, bp, xcent_base, W_l, b_l, W_r, b_r, att, bias, 1024)
    return out.reshape(NBATCH, RATIO, IN_C)
